# Initial kernel scaffold; baseline (speedup 1.0000x reference)
import jax
import jax.numpy as jnp
from jax.experimental import pallas as pl


def kernel(text_emb, demand_kg_emb, x, edge_index, W_fc, eps, W1, b1, g1, be1, W2, b2, gbn, bbn, Wp1, bp1, Wp2, bp2):
    raise NotImplementedError("write your pallas kernel here")


# speedup vs baseline: 2.7602x; 2.7602x over previous
import functools
import math

import jax
import jax.numpy as jnp
from jax import lax
from jax.experimental import pallas as pl
from jax.experimental.pallas import tpu as pltpu
from jax.experimental.pallas import tpu_sc as plsc

N = 10000
E = 160000
FEAT = 256
EMB = 512
HID2 = 2 * EMB
L = 3
K = int(math.ceil(0.5 * N))

NP = 10240
DUMMY = N
AGG_ROWS = 10240
RBLK = 2000
NSUB = 16
NCORE = 2
EB = 128
E_PER_TILE = 10240
E_PAD = NSUB * E_PER_TILE


def _score_body(x_ref, q_ref, o_ref):
    o_ref[...] = jnp.tanh(
        jnp.sum(x_ref[...] * q_ref[...], axis=1, keepdims=True))


def _score_call(x, q2d):
    return pl.pallas_call(
        _score_body,
        out_shape=jax.ShapeDtypeStruct((N, 1), jnp.float32),
    )(x, q2d)


def _select_body(s_ref, mask_ref, sm_ref):
    s = s_ref[...]
    bits = lax.bitcast_convert_type(s, jnp.int32)
    key = bits ^ ((bits >> 31) & jnp.int32(0x7FFFFFFF))
    kb = key ^ jnp.int32(-2147483648)

    kf = jnp.float32(K)

    def tbit(i, lo_u):
        b = 31 - i
        cand = lo_u | (jnp.int32(1) << b)
        t_s = cand ^ jnp.int32(-2147483648)
        cnt = jnp.sum((key >= t_s).astype(jnp.float32))
        return jnp.where(cnt >= kf, cand, lo_u)

    lo_u = lax.fori_loop(0, 32, tbit, jnp.int32(0))
    t_star = lo_u ^ jnp.int32(-2147483648)

    cnt_gt = jnp.sum((key > t_star).astype(jnp.float32))
    r = kf - cnt_gt

    rr = lax.broadcasted_iota(jnp.int32, s.shape, 0)
    cc = lax.broadcasted_iota(jnp.int32, s.shape, 1)
    idx = rr * 128 + cc
    eq = key == t_star

    def mbit(i, lo_m):
        b = 14 - i
        cand = lo_m | (jnp.int32(1) << b)
        ecnt = jnp.sum((eq & (idx < cand)).astype(jnp.float32))
        return jnp.where(ecnt <= r, cand, lo_m)

    m_star = lax.fori_loop(0, 15, mbit, jnp.int32(0))

    mask = (key > t_star) | (eq & (idx < m_star))
    maskf = mask.astype(jnp.float32)
    mask_ref[...] = maskf
    sm_ref[...] = maskf * s
    del kb


def _select_call(scorep):
    return pl.pallas_call(
        _select_body,
        out_shape=[
            jax.ShapeDtypeStruct((NP // 128, 128), jnp.float32),
            jax.ShapeDtypeStruct((NP // 128, 128), jnp.float32),
        ],
    )(scorep)


def _fc_body(x_ref, sm_ref, w_ref, h0, h1, h2, h3, r0, r1, r2, r3):
    z = jnp.dot(x_ref[...] * sm_ref[...], w_ref[...],
                preferred_element_type=jnp.float32)
    zr = jnp.maximum(z, 0.0)
    for c, (hr, rr) in enumerate(((h0, r0), (h1, r1), (h2, r2), (h3, r3))):
        hr[...] = z[:, c * 128:(c + 1) * 128]
        rr[...] = zr[:, c * 128:(c + 1) * 128]


def _fc_call(x, sm_col, w_fc):
    nblk = N // RBLK
    chunk_spec = pl.BlockSpec((RBLK, 128), lambda i: (i, 0))
    return pl.pallas_call(
        _fc_body,
        grid=(nblk,),
        in_specs=[
            pl.BlockSpec((RBLK, FEAT), lambda i: (i, 0)),
            pl.BlockSpec((RBLK, 1), lambda i: (i, 0)),
            pl.BlockSpec((FEAT, EMB), lambda i: (0, 0)),
        ],
        out_specs=[chunk_spec] * 8,
        out_shape=[jax.ShapeDtypeStruct((N, 128), jnp.float32)] * 8,
    )(x, sm_col, w_fc)


def _agg_body(row_hbm, col_hbm, maskt_hbm, zeros_hbm,
              rh0, rh1, rh2, rh3, o0, o1, o2, o3,
              maskt_v, rowb, colb, dstb, rows_v, agg_sh, sem):
    cid = lax.axis_index("c")
    sid = lax.axis_index("s")
    pltpu.sync_copy(maskt_hbm, maskt_v)
    ebase = sid * E_PER_TILE
    nb = E_PER_TILE // EB
    zr = AGG_ROWS // NSUB
    rhs = (rh0, rh1, rh2, rh3)
    outs = (o0, o1, o2, o3)

    for p in range(2):
        for c in range(NCORE):
            chunk = 2 * c + p

            @pl.when(cid == c)
            def _(chunk=chunk):
                rh_t = rhs[chunk]
                out_t = outs[chunk]
                pltpu.sync_copy(zeros_hbm, agg_sh.at[pl.ds(sid * zr, zr)])
                plsc.subcore_barrier()

                def blk(j, carry):
                    off = ebase + j * EB
                    pltpu.sync_copy(row_hbm.at[pl.ds(off, EB)], rowb)
                    pltpu.sync_copy(col_hbm.at[pl.ds(off, EB)], colb)

                    def lane(v, c2):
                        r16 = rowb[pl.ds(v * 16, 16)]
                        c16 = colb[pl.ds(v * 16, 16)]
                        mr = plsc.load_gather(maskt_v, [r16])
                        mc = plsc.load_gather(maskt_v, [c16])
                        keep = (mr > 0) & (mc > 0)
                        dstb[pl.ds(v * 16, 16)] = jnp.where(
                            keep, c16, jnp.int32(DUMMY))
                        return c2

                    lax.fori_loop(0, EB // 16, lane, 0)
                    pltpu.async_copy(rh_t.at[rowb], rows_v, sem).wait()
                    pltpu.sync_copy(rows_v, agg_sh.at[dstb], add=True)
                    return carry

                lax.fori_loop(0, nb, blk, 0)
                plsc.subcore_barrier()
                pltpu.sync_copy(agg_sh.at[pl.ds(sid * zr, zr)],
                                out_t.at[pl.ds(sid * zr, zr)])
                plsc.subcore_barrier()


def _make_agg_call():
    mesh = plsc.VectorSubcoreMesh(core_axis_name="c", subcore_axis_name="s")
    return pl.kernel(
        _agg_body,
        out_type=[jax.ShapeDtypeStruct((AGG_ROWS, 128), jnp.float32)] * 4,
        mesh=mesh,
        scratch_types=[
            pltpu.VMEM((NP,), jnp.int32),
            pltpu.VMEM((EB,), jnp.int32),
            pltpu.VMEM((EB,), jnp.int32),
            pltpu.VMEM((EB,), jnp.int32),
            pltpu.VMEM((EB, 128), jnp.float32),
            pltpu.VMEM_SHARED((AGG_ROWS, 128), jnp.float32),
            pltpu.SemaphoreType.DMA,
        ],
        compiler_params=pltpu.CompilerParams(needs_layout_passes=False),
    )


def _layer_a_body(h0, h1, h2, h3, a0, a1, a2, a3,
                  w_ref, b_ref, mask_ref, eps_ref, z1_ref, st_ref):
    i = pl.program_id(0)
    h = jnp.concatenate([h0[...], h1[...], h2[...], h3[...]], axis=1)
    a = jnp.concatenate([a0[...], a1[...], a2[...], a3[...]], axis=1)
    zin = h * (1.0 + eps_ref[0]) + a
    z1 = jnp.dot(zin, w_ref[...], preferred_element_type=jnp.float32)
    z1 = z1 + b_ref[...]
    z1_ref[...] = z1
    m = mask_ref[...]
    zm = z1 * m

    @pl.when(i == 0)
    def _():
        st_ref[...] = jnp.zeros_like(st_ref)

    st_ref[0:1, :] = st_ref[0:1, :] + jnp.sum(zm, axis=0, keepdims=True)
    st_ref[1:2, :] = st_ref[1:2, :] + jnp.sum(z1 * zm, axis=0, keepdims=True)


def _layer_a_call(hc, ac, w1l, b1l, mask_col, epsl):
    nblk = N // RBLK
    chunk_spec = pl.BlockSpec((RBLK, 128), lambda i: (i, 0))
    return pl.pallas_call(
        _layer_a_body,
        grid=(nblk,),
        in_specs=[chunk_spec] * 8 + [
            pl.BlockSpec((EMB, HID2), lambda i: (0, 0)),
            pl.BlockSpec((1, HID2), lambda i: (0, 0)),
            pl.BlockSpec((RBLK, 1), lambda i: (i, 0)),
            pl.BlockSpec(memory_space=pltpu.SMEM),
        ],
        out_specs=[
            pl.BlockSpec((RBLK, HID2), lambda i: (i, 0)),
            pl.BlockSpec((8, HID2), lambda i: (0, 0)),
        ],
        out_shape=[
            jax.ShapeDtypeStruct((N, HID2), jnp.float32),
            jax.ShapeDtypeStruct((8, HID2), jnp.float32),
        ],
    )(*hc, *ac, w1l, b1l, mask_col, epsl)


def _layer_b_body(z1_ref, st_ref, g_ref, be_ref, w_ref, b_ref, mask_ref,
                  z2_ref, st2_ref):
    i = pl.program_id(0)
    kf = jnp.float32(K)
    mean = st_ref[0:1, :] / kf
    var = st_ref[1:2, :] / kf - mean * mean
    z1 = z1_ref[...]
    xb = g_ref[...] * (z1 - mean) / jnp.sqrt(var + 1e-5) + be_ref[...]
    y = jnp.maximum(xb, 0.0)
    z2 = jnp.dot(y, w_ref[...], preferred_element_type=jnp.float32)
    z2 = z2 + b_ref[...]
    z2_ref[...] = z2
    m = mask_ref[...]
    zm = z2 * m

    @pl.when(i == 0)
    def _():
        st2_ref[...] = jnp.zeros_like(st2_ref)

    st2_ref[0:1, :] = st2_ref[0:1, :] + jnp.sum(zm, axis=0, keepdims=True)
    st2_ref[1:2, :] = st2_ref[1:2, :] + jnp.sum(z2 * zm, axis=0, keepdims=True)


def _layer_b_call(z1, st, g1l, be1l, w2l, b2l, mask_col):
    nblk = N // RBLK
    return pl.pallas_call(
        _layer_b_body,
        grid=(nblk,),
        in_specs=[
            pl.BlockSpec((RBLK, HID2), lambda i: (i, 0)),
            pl.BlockSpec((8, HID2), lambda i: (0, 0)),
            pl.BlockSpec((1, HID2), lambda i: (0, 0)),
            pl.BlockSpec((1, HID2), lambda i: (0, 0)),
            pl.BlockSpec((HID2, EMB), lambda i: (0, 0)),
            pl.BlockSpec((1, EMB), lambda i: (0, 0)),
            pl.BlockSpec((RBLK, 1), lambda i: (i, 0)),
        ],
        out_specs=[
            pl.BlockSpec((RBLK, EMB), lambda i: (i, 0)),
            pl.BlockSpec((8, EMB), lambda i: (0, 0)),
        ],
        out_shape=[
            jax.ShapeDtypeStruct((N, EMB), jnp.float32),
            jax.ShapeDtypeStruct((8, EMB), jnp.float32),
        ],
    )(z1, st, g1l, be1l, w2l, b2l, mask_col)


def _layer_c_body(z2_ref, st_ref, g_ref, be_ref, h0, h1, h2, h3):
    kf = jnp.float32(K)
    mean = st_ref[0:1, :] / kf
    var = st_ref[1:2, :] / kf - mean * mean
    z2 = z2_ref[...]
    xb = g_ref[...] * (z2 - mean) / jnp.sqrt(var + 1e-5) + be_ref[...]
    h = jnp.maximum(xb, 0.0)
    for c, hr in enumerate((h0, h1, h2, h3)):
        hr[...] = h[:, c * 128:(c + 1) * 128]


def _layer_c_call(z2, st2, gbnl, bbnl):
    nblk = N // RBLK
    chunk_spec = pl.BlockSpec((RBLK, 128), lambda i: (i, 0))
    return pl.pallas_call(
        _layer_c_body,
        grid=(nblk,),
        in_specs=[
            pl.BlockSpec((RBLK, EMB), lambda i: (i, 0)),
            pl.BlockSpec((8, EMB), lambda i: (0, 0)),
            pl.BlockSpec((1, EMB), lambda i: (0, 0)),
            pl.BlockSpec((1, EMB), lambda i: (0, 0)),
        ],
        out_specs=[chunk_spec] * 4,
        out_shape=[jax.ShapeDtypeStruct((N, 128), jnp.float32)] * 4,
    )(z2, st2, gbnl, bbnl)


def _layer_pool_body(z2_ref, st_ref, g_ref, be_ref, mask_ref, pool_ref):
    i = pl.program_id(0)
    kf = jnp.float32(K)
    mean = st_ref[0:1, :] / kf
    var = st_ref[1:2, :] / kf - mean * mean
    z2 = z2_ref[...]
    h = g_ref[...] * (z2 - mean) / jnp.sqrt(var + 1e-5) + be_ref[...]

    @pl.when(i == 0)
    def _():
        pool_ref[...] = jnp.zeros_like(pool_ref)

    pool_ref[0:1, :] = pool_ref[0:1, :] + jnp.sum(
        h * mask_ref[...], axis=0, keepdims=True)


def _layer_pool_call(z2, st2, gbnl, bbnl, mask_col):
    nblk = N // RBLK
    return pl.pallas_call(
        _layer_pool_body,
        grid=(nblk,),
        in_specs=[
            pl.BlockSpec((RBLK, EMB), lambda i: (i, 0)),
            pl.BlockSpec((8, EMB), lambda i: (0, 0)),
            pl.BlockSpec((1, EMB), lambda i: (0, 0)),
            pl.BlockSpec((1, EMB), lambda i: (0, 0)),
            pl.BlockSpec((RBLK, 1), lambda i: (i, 0)),
        ],
        out_specs=pl.BlockSpec((8, EMB), lambda i: (0, 0)),
        out_shape=jax.ShapeDtypeStruct((8, EMB), jnp.float32),
    )(z2, st2, gbnl, bbnl, mask_col)


def _head_body(pool_ref, text_ref, wp1_ref, bp1_ref, wp2_ref, bp2_ref, o_ref):
    p = pool_ref[0:1, :] * (1.0 / jnp.float32(K))
    a1 = wp1_ref[0:EMB, :]
    a2 = wp1_ref[EMB:EMB + FEAT, :]
    r = jnp.dot(p, a1, preferred_element_type=jnp.float32)
    r = r + jnp.dot(text_ref[...], a2, preferred_element_type=jnp.float32)
    r = jnp.maximum(r + bp1_ref[...], 0.0)
    o = jnp.dot(r, wp2_ref[...], preferred_element_type=jnp.float32)
    o_ref[...] = o + bp2_ref[...]


def _head_call(pool, text_emb, wp1, bp1, wp2, bp2):
    return pl.pallas_call(
        _head_body,
        out_shape=jax.ShapeDtypeStruct((1, 2), jnp.float32),
    )(pool, text_emb, wp1, bp1, wp2, bp2)


def kernel(text_emb, demand_kg_emb, x, edge_index, W_fc, eps, W1, b1, g1,
           be1, W2, b2, gbn, bbn, Wp1, bp1, Wp2, bp2):
    score = _score_call(x, demand_kg_emb)
    scorep = jnp.concatenate(
        [score.reshape(N), jnp.full((NP - N,), -2.0, jnp.float32)]
    ).reshape(NP // 128, 128)
    maskf, smf = _select_call(scorep)
    mask_flat = maskf.reshape(NP)
    mask_col = mask_flat[:N].reshape(N, 1)
    maskt = mask_flat.astype(jnp.int32)
    sm_col = smf.reshape(NP)[:N].reshape(N, 1)

    outs = _fc_call(x, sm_col, W_fc)
    hc, rhc = list(outs[:4]), list(outs[4:])

    row = edge_index[0]
    col = edge_index[1]
    pad = jnp.zeros((E_PAD - E,), jnp.int32)
    rowp = jnp.concatenate([row.astype(jnp.int32), pad])
    colp = jnp.concatenate([col.astype(jnp.int32), pad + jnp.int32(DUMMY)])
    zeros_sc = jnp.zeros((AGG_ROWS // NSUB, 128), jnp.float32)

    agg_fn = _make_agg_call()

    for l in range(L):
        ac = agg_fn(rowp, colp, maskt, zeros_sc, *rhc)
        z1, st1 = _layer_a_call(
            hc, ac, W1[l], b1[l].reshape(1, HID2), mask_col,
            eps[l].reshape(1))
        z2, st2 = _layer_b_call(
            z1, st1, g1[l].reshape(1, HID2), be1[l].reshape(1, HID2),
            W2[l], b2[l].reshape(1, EMB), mask_col)
        if l < L - 1:
            hc = _layer_c_call(z2, st2, gbn[l].reshape(1, EMB),
                               bbn[l].reshape(1, EMB))
            rhc = hc
        else:
            pool = _layer_pool_call(z2, st2, gbn[l].reshape(1, EMB),
                                    bbn[l].reshape(1, EMB), mask_col)

    return _head_call(pool, text_emb, Wp1, bp1.reshape(1, -1), Wp2,
                      bp2.reshape(1, 2))
